# baseline (speedup 1.0000x reference)
import functools

import jax
import jax.numpy as jnp
from jax import lax
from jax.experimental import pallas as pl
from jax.experimental.pallas import tpu as pltpu
from jax.experimental.pallas import tpu_sc as plsc

N = 16384
D = 64
DH = D // 2
NT = 16
CHUNK = 512
IDXW = 128
NSUB = CHUNK // IDXW
SUP = 8
SROWS = SUP * CHUNK // IDXW


def _sc_body(nsup, inp_hbm, p_hbm, dflat_hbm, out_hbm,
             pbuf, gat, acc, sem_p, sem_g, sem_s):
    c = lax.axis_index("c")
    s = lax.axis_index("s")
    tile_rows = nsup * SROWS

    @plsc.parallel_loop(0, CHUNK, unroll=4)
    def _zb(i):
        gat[0, i, pl.ds(0, 16)] = jnp.zeros((16,), jnp.float32)
        gat[0, i, pl.ds(16, 16)] = jnp.zeros((16,), jnp.float32)
    arows = N // NT
    pltpu.sync_copy(gat.at[0], acc.at[pl.ds(s * arows, CHUNK)])
    pltpu.sync_copy(gat.at[0], acc.at[pl.ds(s * arows + CHUNK, CHUNK)])
    plsc.subcore_barrier()

    def _p_slice(u):
        return p_hbm.at[c, pl.ds(s * tile_rows + u * SROWS, SROWS)]

    def _sup(u, b):
        def fire_gather(k, g):
            for j in range(NSUB):
                pltpu.async_copy(dflat_hbm.at[pbuf.at[b, k * NSUB + j, 0]],
                                 gat.at[g, pl.ds(j * IDXW, IDXW)], sem_g.at[g])

        def wait_scat(g):
            for j in range(NSUB):
                pltpu.make_async_copy(gat.at[g, pl.ds(j * IDXW, IDXW)],
                                      acc.at[pbuf.at[b, j, 1]],
                                      sem_s.at[g]).wait()

        def scale_scatter(k, g):
            for j in range(NSUB):
                pltpu.make_async_copy(dflat_hbm.at[pbuf.at[b, k * NSUB + j, 0]],
                                      gat.at[g, pl.ds(j * IDXW, IDXW)],
                                      sem_g.at[g]).wait()

            @plsc.parallel_loop(0, CHUNK // 16, unroll=4)
            def _mul(i):
                r = k * NSUB + i // 8
                l = (i % 8) * 16
                vv = plsc.bitcast(pbuf[b, r, 2, pl.ds(l, 16)], jnp.float32)
                for t in range(16):
                    q = i * 16 + t
                    gat[g, q, pl.ds(0, 16)] = gat[g, q, pl.ds(0, 16)] * vv[t]
                    gat[g, q, pl.ds(16, 16)] = gat[g, q, pl.ds(16, 16)] * vv[t]

            for j in range(NSUB):
                pltpu.async_copy(gat.at[g, pl.ds(j * IDXW, IDXW)],
                                 acc.at[pbuf.at[b, k * NSUB + j, 1]],
                                 sem_s.at[g], add=True)

        pltpu.make_async_copy(_p_slice(u), pbuf.at[b], sem_p.at[b]).wait()
        un = jnp.minimum(u + 1, nsup - 1)
        pltpu.async_copy(_p_slice(un), pbuf.at[1 - b], sem_p.at[1 - b])

        fire_gather(0, 0)

        def _kp(kp, _):
            k0 = kp * 2

            @pl.when(kp > 0)
            def _():
                wait_scat(1)
            fire_gather(k0 + 1, 1)
            scale_scatter(k0, 0)
            scale_scatter(k0 + 1, 1)

            @pl.when(kp < SUP // 2 - 1)
            def _():
                wait_scat(0)
                fire_gather(k0 + 2, 0)
            return _
        lax.fori_loop(0, SUP // 2, _kp, None)
        wait_scat(0)
        wait_scat(1)

    pltpu.async_copy(_p_slice(0), pbuf.at[0], sem_p.at[0])

    def _pair(u2, _):
        _sup(2 * u2, 0)
        _sup(2 * u2 + 1, 1)
        return _
    lax.fori_loop(0, nsup // 2, _pair, None)
    if nsup % 2:
        _sup(nsup - 1, 0)
    last_pend = 1 - ((nsup - 1) % 2)
    pltpu.make_async_copy(_p_slice(nsup - 1), pbuf.at[last_pend],
                          sem_p.at[last_pend]).wait()

    plsc.subcore_barrier()

    for half in range(2):
        r0 = s * arows + half * CHUNK
        pltpu.sync_copy(inp_hbm.at[c, pl.ds(r0, CHUNK)], gat.at[0])
        pltpu.sync_copy(acc.at[pl.ds(r0, CHUNK)], gat.at[1])

        @plsc.parallel_loop(0, CHUNK, unroll=4)
        def _add(i):
            gat[0, i, pl.ds(0, 16)] = gat[0, i, pl.ds(0, 16)] + gat[1, i, pl.ds(0, 16)]
            gat[0, i, pl.ds(16, 16)] = gat[0, i, pl.ds(16, 16)] + gat[1, i, pl.ds(16, 16)]
        pltpu.sync_copy(gat.at[0], out_hbm.at[c, pl.ds(r0, CHUNK)])


def kernel(input_mat, sparse_indices, sparse_values, dense):
    nnz = sparse_values.shape[0]
    quantum = NT * SUP * CHUNK
    nnz_pad = ((nnz + quantum - 1) // quantum) * quantum
    nsup = nnz_pad // quantum
    pad = nnz_pad - nnz
    ar = jnp.arange(pad, dtype=jnp.int32)
    rows_p = jnp.concatenate([sparse_indices[0], (ar * 97) % N])
    cols_p = jnp.concatenate([sparse_indices[1], (ar * 89) % N])
    vals_p = jnp.pad(sparse_values, (0, pad))
    vbits = lax.bitcast_convert_type(vals_p, jnp.int32).reshape(-1, IDXW)
    rows2 = rows_p.reshape(-1, IDXW)
    pmeta = jnp.stack([
        jnp.stack([cols_p.reshape(-1, IDXW) + cc * N, rows2, vbits], axis=1)
        for cc in range(2)])
    dflat = jnp.concatenate([dense[:, :DH], dense[:, DH:]], axis=0)
    inp2 = jnp.stack([input_mat[:, :DH], input_mat[:, DH:]])

    mesh = plsc.VectorSubcoreMesh(core_axis_name="c", subcore_axis_name="s")
    body = functools.partial(_sc_body, nsup)
    out2 = pl.kernel(
        body,
        out_type=jax.ShapeDtypeStruct((2, N, DH), jnp.float32),
        mesh=mesh,
        compiler_params=pltpu.CompilerParams(use_tc_tiling_on_sc=False,
                                             needs_layout_passes=False),
        scratch_types=[
            pltpu.VMEM((2, SROWS, 3, IDXW), jnp.int32),
            pltpu.VMEM((2, CHUNK, DH), jnp.float32),
            pltpu.VMEM_SHARED((N, DH), jnp.float32),
            pltpu.SemaphoreType.DMA((2,)),
            pltpu.SemaphoreType.DMA((2,)),
            pltpu.SemaphoreType.DMA((2,)),
        ],
    )(inp2, pmeta, dflat)
    return jnp.concatenate([out2[0], out2[1]], axis=1)


# speedup vs baseline: 1.0094x; 1.0094x over previous
import functools

import jax
import jax.numpy as jnp
from jax import lax
from jax.experimental import pallas as pl
from jax.experimental.pallas import tpu as pltpu
from jax.experimental.pallas import tpu_sc as plsc

N = 16384
D = 64
DH = D // 2
NT = 16
CHUNK = 512
IDXW = 128
NSUB = CHUNK // IDXW
SUP = 8
SROWS = SUP * CHUNK // IDXW


def _sc_body(nsup, inp_hbm, p_hbm, dflat_hbm, out_hbm,
             pbuf, gat, acc, sem_p, sem_g, sem_s):
    c = lax.axis_index("c")
    s = lax.axis_index("s")
    tile_rows = nsup * SROWS

    @plsc.parallel_loop(0, CHUNK, unroll=4)
    def _zb(i):
        gat[0, i, pl.ds(0, 16)] = jnp.zeros((16,), jnp.float32)
        gat[0, i, pl.ds(16, 16)] = jnp.zeros((16,), jnp.float32)
    arows = N // NT
    pltpu.sync_copy(gat.at[0], acc.at[pl.ds(s * arows, CHUNK)])
    pltpu.sync_copy(gat.at[0], acc.at[pl.ds(s * arows + CHUNK, CHUNK)])
    plsc.subcore_barrier()

    def _p_slice(u):
        return p_hbm.at[c, pl.ds(s * tile_rows + u * SROWS, SROWS)]

    def _sup(u, b):
        pltpu.make_async_copy(_p_slice(u), pbuf.at[b], sem_p.at[b]).wait()
        un = jnp.minimum(u + 1, nsup - 1)
        pltpu.async_copy(_p_slice(un), pbuf.at[1 - b], sem_p.at[1 - b])

        pend_g, pend_s = {}, {}

        def fire_gather(k):
            g = k % 2
            pend_g[k] = [
                pltpu.async_copy(dflat_hbm.at[pbuf.at[b, k * NSUB + j, 0]],
                                 gat.at[g, pl.ds(j * IDXW, IDXW)], sem_g.at[g])
                for j in range(NSUB)]

        def scale_scatter(k):
            g = k % 2
            for cp in pend_g.pop(k):
                cp.wait()

            @plsc.parallel_loop(0, CHUNK // 16, unroll=4)
            def _mul(i):
                r = k * NSUB + i // 8
                l = (i % 8) * 16
                vv = plsc.bitcast(pbuf[b, r, 2, pl.ds(l, 16)], jnp.float32)
                for t in range(16):
                    q = i * 16 + t
                    gat[g, q, pl.ds(0, 16)] = gat[g, q, pl.ds(0, 16)] * vv[t]
                    gat[g, q, pl.ds(16, 16)] = gat[g, q, pl.ds(16, 16)] * vv[t]

            pend_s[k] = [
                pltpu.async_copy(gat.at[g, pl.ds(j * IDXW, IDXW)],
                                 acc.at[pbuf.at[b, k * NSUB + j, 1]],
                                 sem_s.at[g], add=True)
                for j in range(NSUB)]

        for k in range(SUP):
            if k >= 2:
                for cp in pend_s.pop(k - 2):
                    cp.wait()
            fire_gather(k)
            if k >= 1:
                scale_scatter(k - 1)
        scale_scatter(SUP - 1)
        for kk in (SUP - 2, SUP - 1):
            for cp in pend_s.pop(kk):
                cp.wait()

    pltpu.async_copy(_p_slice(0), pbuf.at[0], sem_p.at[0])

    def _pair(u2, _):
        _sup(2 * u2, 0)
        _sup(2 * u2 + 1, 1)
        return _
    lax.fori_loop(0, nsup // 2, _pair, None)
    if nsup % 2:
        _sup(nsup - 1, 0)
    last_pend = 1 - ((nsup - 1) % 2)
    pltpu.make_async_copy(_p_slice(nsup - 1), pbuf.at[last_pend],
                          sem_p.at[last_pend]).wait()

    plsc.subcore_barrier()

    for half in range(2):
        r0 = s * arows + half * CHUNK
        pltpu.sync_copy(inp_hbm.at[c, pl.ds(r0, CHUNK)], gat.at[0])
        pltpu.sync_copy(acc.at[pl.ds(r0, CHUNK)], gat.at[1])

        @plsc.parallel_loop(0, CHUNK, unroll=4)
        def _add(i):
            gat[0, i, pl.ds(0, 16)] = gat[0, i, pl.ds(0, 16)] + gat[1, i, pl.ds(0, 16)]
            gat[0, i, pl.ds(16, 16)] = gat[0, i, pl.ds(16, 16)] + gat[1, i, pl.ds(16, 16)]
        pltpu.sync_copy(gat.at[0], out_hbm.at[c, pl.ds(r0, CHUNK)])


def kernel(input_mat, sparse_indices, sparse_values, dense):
    nnz = sparse_values.shape[0]
    quantum = NT * SUP * CHUNK
    nnz_pad = ((nnz + quantum - 1) // quantum) * quantum
    nsup = nnz_pad // quantum
    pad = nnz_pad - nnz
    ar = jnp.arange(pad, dtype=jnp.int32)
    rows_p = jnp.concatenate([sparse_indices[0], (ar * 97) % N])
    cols_p = jnp.concatenate([sparse_indices[1], (ar * 89) % N])
    vals_p = jnp.pad(sparse_values, (0, pad))
    vbits = lax.bitcast_convert_type(vals_p, jnp.int32).reshape(-1, IDXW)
    rows2 = rows_p.reshape(-1, IDXW)
    pmeta = jnp.stack([
        jnp.stack([cols_p.reshape(-1, IDXW) + cc * N, rows2, vbits], axis=1)
        for cc in range(2)])
    dflat = jnp.concatenate([dense[:, :DH], dense[:, DH:]], axis=0)
    inp2 = jnp.stack([input_mat[:, :DH], input_mat[:, DH:]])

    mesh = plsc.VectorSubcoreMesh(core_axis_name="c", subcore_axis_name="s")
    body = functools.partial(_sc_body, nsup)
    out2 = pl.kernel(
        body,
        out_type=jax.ShapeDtypeStruct((2, N, DH), jnp.float32),
        mesh=mesh,
        compiler_params=pltpu.CompilerParams(use_tc_tiling_on_sc=False,
                                             needs_layout_passes=False),
        scratch_types=[
            pltpu.VMEM((2, SROWS, 3, IDXW), jnp.int32),
            pltpu.VMEM((2, CHUNK, DH), jnp.float32),
            pltpu.VMEM_SHARED((N, DH), jnp.float32),
            pltpu.SemaphoreType.DMA((2,)),
            pltpu.SemaphoreType.DMA((2,)),
            pltpu.SemaphoreType.DMA((2,)),
        ],
    )(inp2, pmeta, dflat)
    return jnp.concatenate([out2[0], out2[1]], axis=1)
